# baseline (speedup 1.0000x reference)
import functools

import jax
import jax.numpy as jnp
from jax import lax
from jax.experimental import pallas as pl
from jax.experimental.pallas import tpu as pltpu
from jax.experimental.pallas import tpu_sc as plsc

_NC = 2
_NS = 16
_CH = 128


@functools.lru_cache(maxsize=None)
def _make_sc_agg(N, NCH, with_deg):
  ch = _CH
  rt = N // _NS
  assert N % (_NS * 8) == 0 and NCH % 2 == 0

  mesh = plsc.VectorSubcoreMesh(
      core_axis_name="c", subcore_axis_name="s",
      num_cores=_NC, num_subcores=_NS)

  out_type = [jax.ShapeDtypeStruct((_NC * N, ch), jnp.float32)]
  scratch = [
      [pltpu.VMEM((ch,), jnp.int32) for _ in range(2)],
      [pltpu.VMEM((ch,), jnp.int32) for _ in range(2)],
      [pltpu.VMEM((ch, ch), jnp.float32) for _ in range(2)],
      [pltpu.SemaphoreType.DMA for _ in range(2)],
      [pltpu.SemaphoreType.DMA for _ in range(2)],
      pltpu.VMEM_SHARED((N, ch), jnp.float32),
  ]
  if with_deg:
    out_type.append(jax.ShapeDtypeStruct((_NC * N,), jnp.float32))
    scratch.append(pltpu.VMEM_SHARED((N,), jnp.float32))
    scratch.append(pltpu.VMEM((ch,), jnp.float32))

  @functools.partial(
      pl.kernel,
      mesh=mesh,
      compiler_params=pltpu.CompilerParams(use_tc_tiling_on_sc=False),
      out_type=out_type,
      scratch_types=scratch,
  )
  def sc_agg(feat_hbm, src_hbm, dst_hbm, zero2_hbm, zero1_hbm, *refs):
    if with_deg:
      (out_hbm, deg_hbm, sbuf, dbuf, rows, isems, gsems, acc, dacc,
       ones_v) = refs
    else:
      out_hbm, sbuf, dbuf, rows, isems, gsems, acc = refs

    c = lax.axis_index("c")
    s = lax.axis_index("s")
    r0 = s * rt
    pltpu.sync_copy(zero2_hbm.at[pl.ds(r0, rt)], acc.at[pl.ds(r0, rt)])
    if with_deg:
      pltpu.sync_copy(zero1_hbm.at[pl.ds(r0, rt)], dacc.at[pl.ds(r0, rt)])
      for i in range(ch // 16):
        ones_v[pl.ds(i * 16, 16)] = jnp.ones((16,), jnp.float32)
    plsc.subcore_barrier()

    wid = c * _NS + s

    def idx_copies(k, b):
      return (pltpu.make_async_copy(src_hbm.at[wid, k], sbuf[b], isems[b]),
              pltpu.make_async_copy(dst_hbm.at[wid, k], dbuf[b], isems[b]))

    def gather_copy(b):
      return pltpu.make_async_copy(feat_hbm.at[sbuf[b]], rows[b], gsems[b])

    for cp in idx_copies(0, 0):
      cp.start()
      cp.wait()
    for cp in idx_copies(1, 1):
      cp.start()
    gather_copy(0).start()

    def turn(k, b):
      @pl.when(k + 1 < NCH)
      def _():
        for cp in idx_copies(k + 1, 1 - b):
          cp.wait()
        gather_copy(1 - b).start()

      gather_copy(b).wait()
      if with_deg:
        pltpu.sync_copy(ones_v, dacc.at[dbuf[b]], add=True)
      pltpu.sync_copy(rows[b], acc.at[dbuf[b]], add=True)

      @pl.when(k + 2 < NCH)
      def _():
        for cp in idx_copies(k + 2, b):
          cp.start()

    def body(o, carry):
      turn(2 * o, 0)
      turn(2 * o + 1, 1)
      return carry

    lax.fori_loop(0, NCH // 2, body, 0)
    plsc.subcore_barrier()
    pltpu.sync_copy(acc.at[pl.ds(r0, rt)],
                    out_hbm.at[pl.ds(c * N + r0, rt)])
    if with_deg:
      pltpu.sync_copy(dacc.at[pl.ds(r0, rt)],
                      deg_hbm.at[pl.ds(c * N + r0, rt)])

  return sc_agg


def _tc_layer1(x, p0, p1, degt, w_self, w_neigh, b, n_real):
  n, d = x.shape
  bn = 1264
  assert n % bn == 0

  def body(x_ref, p0_ref, p1_ref, dg_ref, ws_ref, wn_ref, b_ref,
           h_ref, dinv_ref):
    dg = dg_ref[...]
    dinv = 1.0 / jnp.maximum(dg[:, 0:1] + dg[:, 1:2], 1.0)
    agg = (p0_ref[...] + p1_ref[...]) * dinv
    h = (jnp.dot(x_ref[...], ws_ref[...], preferred_element_type=jnp.float32)
         + jnp.dot(agg, wn_ref[...], preferred_element_type=jnp.float32)
         + b_ref[...])
    rowid = pl.program_id(0) * bn + lax.broadcasted_iota(jnp.int32, (bn, 1), 0)
    h_ref[...] = jnp.where(rowid < n_real, jnp.maximum(h, 0.0), 0.0)
    dinv_ref[...] = jnp.broadcast_to(dinv, (bn, d))

  return pl.pallas_call(
      body,
      grid=(n // bn,),
      in_specs=[
          pl.BlockSpec((bn, d), lambda i: (i, 0)),
          pl.BlockSpec((bn, d), lambda i: (i, 0)),
          pl.BlockSpec((bn, d), lambda i: (i, 0)),
          pl.BlockSpec((bn, 2), lambda i: (i, 0)),
          pl.BlockSpec((d, d), lambda i: (0, 0)),
          pl.BlockSpec((d, d), lambda i: (0, 0)),
          pl.BlockSpec((1, d), lambda i: (0, 0)),
      ],
      out_specs=[pl.BlockSpec((bn, d), lambda i: (i, 0)),
                 pl.BlockSpec((bn, d), lambda i: (i, 0))],
      out_shape=[jax.ShapeDtypeStruct((n, d), jnp.float32),
                 jax.ShapeDtypeStruct((n, d), jnp.float32)],
  )(x, p0, p1, degt, w_self, w_neigh, b.reshape(1, d))


def _tc_layer2(h, q0, q1, dinv, w_self, w_neigh, b, w_fc, b_fc):
  n, d = h.shape
  co = w_fc.shape[1]
  bn = 1264
  assert n % bn == 0

  def body(h_ref, q0_ref, q1_ref, dinv_ref, ws_ref, wn_ref, b_ref,
           wfc_ref, bfc_ref, logits_ref, h2_ref):
    agg = (q0_ref[...] + q1_ref[...]) * dinv_ref[...]
    h2 = (jnp.dot(h_ref[...], ws_ref[...], preferred_element_type=jnp.float32)
          + jnp.dot(agg, wn_ref[...], preferred_element_type=jnp.float32)
          + b_ref[...])
    h2_ref[...] = h2
    logits_ref[...] = (
        jnp.dot(h2, wfc_ref[...], preferred_element_type=jnp.float32)
        + bfc_ref[...])

  return pl.pallas_call(
      body,
      grid=(n // bn,),
      in_specs=[
          pl.BlockSpec((bn, d), lambda i: (i, 0)),
          pl.BlockSpec((bn, d), lambda i: (i, 0)),
          pl.BlockSpec((bn, d), lambda i: (i, 0)),
          pl.BlockSpec((bn, d), lambda i: (i, 0)),
          pl.BlockSpec((d, d), lambda i: (0, 0)),
          pl.BlockSpec((d, d), lambda i: (0, 0)),
          pl.BlockSpec((1, d), lambda i: (0, 0)),
          pl.BlockSpec((d, co), lambda i: (0, 0)),
          pl.BlockSpec((1, co), lambda i: (0, 0)),
      ],
      out_specs=[pl.BlockSpec((bn, co), lambda i: (i, 0)),
                 pl.BlockSpec((bn, d), lambda i: (i, 0))],
      out_shape=[jax.ShapeDtypeStruct((n, co), jnp.float32),
                 jax.ShapeDtypeStruct((n, d), jnp.float32)],
  )(h, q0, q1, dinv, w_self, w_neigh, b.reshape(1, d), w_fc,
    b_fc.reshape(1, co))


def kernel(x, edge_index, W_self1, W_neigh1, b1, W_self2, W_neigh2, b2,
           W_fc, b_fc):
  n, d = x.shape
  e = edge_index.shape[1]
  nwk = _NC * _NS
  n2 = ((n + _NS * 8 - 1) // (_NS * 8)) * (_NS * 8)

  nch = -(-e // (nwk * _CH))
  nch += nch % 2
  ep = nwk * nch * _CH
  src = jnp.concatenate(
      [edge_index[0].astype(jnp.int32),
       jnp.full((ep - e,), n, jnp.int32)]).reshape(nwk, nch, _CH)
  dst = jnp.concatenate(
      [edge_index[1].astype(jnp.int32),
       jnp.full((ep - e,), n, jnp.int32)]).reshape(nwk, nch, _CH)

  xpad = jnp.concatenate(
      [x, jnp.zeros((n2 - n, d), jnp.float32)], axis=0)
  zeros2 = jnp.zeros((n2, d), jnp.float32)
  zeros1 = jnp.zeros((n2,), jnp.float32)

  p, deg = _make_sc_agg(n2, nch, True)(xpad, src, dst, zeros2, zeros1)
  degt = deg.reshape(_NC, n2).T
  h, dinv = _tc_layer1(xpad, p[:n2], p[n2:], degt, W_self1, W_neigh1, b1, n)
  (q,) = _make_sc_agg(n2, nch, False)(h, src, dst, zeros2, zeros1)
  logits, h2 = _tc_layer2(h, q[:n2], q[n2:], dinv, W_self2, W_neigh2, b2,
                          W_fc, b_fc)
  return (logits[:n], h2[:n])


# speedup vs baseline: 1.1284x; 1.1284x over previous
import functools

import jax
import jax.numpy as jnp
from jax import lax
from jax.experimental import pallas as pl
from jax.experimental.pallas import tpu as pltpu
from jax.experimental.pallas import tpu_sc as plsc

_NC = 2
_NS = 16
_CH = 128


@functools.lru_cache(maxsize=None)
def _make_sc_agg(N, NCH, with_deg):
  ch = _CH
  rt = N // _NS
  assert N % (_NS * 8) == 0 and NCH % 2 == 0

  mesh = plsc.VectorSubcoreMesh(
      core_axis_name="c", subcore_axis_name="s",
      num_cores=_NC, num_subcores=_NS)

  out_type = [jax.ShapeDtypeStruct((_NC * N, ch), jnp.float32)]
  scratch = [
      [pltpu.VMEM((ch,), jnp.int32) for _ in range(2)],
      [pltpu.VMEM((ch,), jnp.int32) for _ in range(2)],
      [pltpu.VMEM((ch, ch), jnp.float32) for _ in range(2)],
      [pltpu.SemaphoreType.DMA for _ in range(2)],
      [pltpu.SemaphoreType.DMA for _ in range(2)],
      pltpu.VMEM_SHARED((N, ch), jnp.float32),
  ]
  if with_deg:
    out_type.append(jax.ShapeDtypeStruct((_NC * N,), jnp.float32))
    scratch.append(pltpu.VMEM_SHARED((N,), jnp.float32))
    scratch.append(pltpu.VMEM((ch,), jnp.float32))

  @functools.partial(
      pl.kernel,
      mesh=mesh,
      compiler_params=pltpu.CompilerParams(use_tc_tiling_on_sc=False),
      out_type=out_type,
      scratch_types=scratch,
  )
  def sc_agg(feat_hbm, src_hbm, dst_hbm, zero2_hbm, zero1_hbm, *refs):
    if with_deg:
      (out_hbm, deg_hbm, sbuf, dbuf, rows, isems, gsems, acc, dacc,
       ones_v) = refs
    else:
      out_hbm, sbuf, dbuf, rows, isems, gsems, acc = refs

    c = lax.axis_index("c")
    s = lax.axis_index("s")
    r0 = s * rt
    pltpu.sync_copy(zero2_hbm.at[pl.ds(r0, rt)], acc.at[pl.ds(r0, rt)])
    if with_deg:
      pltpu.sync_copy(zero1_hbm.at[pl.ds(r0, rt)], dacc.at[pl.ds(r0, rt)])
      for i in range(ch // 16):
        ones_v[pl.ds(i * 16, 16)] = jnp.ones((16,), jnp.float32)
    plsc.subcore_barrier()

    wid = c * _NS + s

    def idx_copies(k, b):
      return (pltpu.make_async_copy(src_hbm.at[wid, k], sbuf[b], isems[b]),
              pltpu.make_async_copy(dst_hbm.at[wid, k], dbuf[b], isems[b]))

    def gather_copy(b):
      return pltpu.make_async_copy(feat_hbm.at[sbuf[b]], rows[b], gsems[b])

    for cp in idx_copies(0, 0):
      cp.start()
      cp.wait()
    for cp in idx_copies(1, 1):
      cp.start()
    gather_copy(0).start()

    def turn(k, b):
      @pl.when(k + 1 < NCH)
      def _():
        for cp in idx_copies(k + 1, 1 - b):
          cp.wait()
        gather_copy(1 - b).start()

      gather_copy(b).wait()
      if with_deg:
        pltpu.sync_copy(ones_v, dacc.at[dbuf[b]], add=True)
      pltpu.sync_copy(rows[b], acc.at[dbuf[b]], add=True)

      @pl.when(k + 2 < NCH)
      def _():
        for cp in idx_copies(k + 2, b):
          cp.start()

    def body(o, carry):
      turn(2 * o, 0)
      turn(2 * o + 1, 1)
      return carry

    lax.fori_loop(0, NCH // 2, body, 0)
    plsc.subcore_barrier()
    pltpu.sync_copy(acc.at[pl.ds(r0, rt)],
                    out_hbm.at[pl.ds(c * N + r0, rt)])
    if with_deg:
      pltpu.sync_copy(dacc.at[pl.ds(r0, rt)],
                      deg_hbm.at[pl.ds(c * N + r0, rt)])

  return sc_agg


def _tc_layer1(x, p0, p1, degt, w_self, w_neigh, b, n_real):
  n, d = x.shape
  bn = 1264
  assert n % bn == 0

  def body(x_ref, p0_ref, p1_ref, dg_ref, ws_ref, wn_ref, b_ref,
           h_ref, dinv_ref):
    dg = dg_ref[...]
    dinv = 1.0 / jnp.maximum(dg[:, 0:1] + dg[:, 1:2], 1.0)
    agg = (p0_ref[...] + p1_ref[...]) * dinv
    h = (jnp.dot(x_ref[...], ws_ref[...], preferred_element_type=jnp.float32)
         + jnp.dot(agg, wn_ref[...], preferred_element_type=jnp.float32)
         + b_ref[...])
    rowid = pl.program_id(0) * bn + lax.broadcasted_iota(jnp.int32, (bn, 1), 0)
    h_ref[...] = jnp.where(rowid < n_real, jnp.maximum(h, 0.0), 0.0)
    dinv_ref[...] = jnp.broadcast_to(dinv, (bn, d))

  return pl.pallas_call(
      body,
      grid=(n // bn,),
      in_specs=[
          pl.BlockSpec((bn, d), lambda i: (i, 0)),
          pl.BlockSpec((bn, d), lambda i: (i, 0)),
          pl.BlockSpec((bn, d), lambda i: (i, 0)),
          pl.BlockSpec((bn, 2), lambda i: (i, 0)),
          pl.BlockSpec((d, d), lambda i: (0, 0)),
          pl.BlockSpec((d, d), lambda i: (0, 0)),
          pl.BlockSpec((1, d), lambda i: (0, 0)),
      ],
      out_specs=[pl.BlockSpec((bn, d), lambda i: (i, 0)),
                 pl.BlockSpec((bn, d), lambda i: (i, 0))],
      out_shape=[jax.ShapeDtypeStruct((n, d), jnp.float32),
                 jax.ShapeDtypeStruct((n, d), jnp.float32)],
  )(x, p0, p1, degt, w_self, w_neigh, b.reshape(1, d))


def _tc_layer2(h, q0, q1, dinv, w_self, w_neigh, b, w_fc, b_fc):
  n, d = h.shape
  co = w_fc.shape[1]
  bn = 1264
  assert n % bn == 0

  def body(h_ref, q0_ref, q1_ref, dinv_ref, ws_ref, wn_ref, b_ref,
           wfc_ref, bfc_ref, logits_ref, h2_ref):
    agg = (q0_ref[...] + q1_ref[...]) * dinv_ref[...]
    h2 = (jnp.dot(h_ref[...], ws_ref[...], preferred_element_type=jnp.float32)
          + jnp.dot(agg, wn_ref[...], preferred_element_type=jnp.float32)
          + b_ref[...])
    h2_ref[...] = h2
    logits_ref[...] = (
        jnp.dot(h2, wfc_ref[...], preferred_element_type=jnp.float32)
        + bfc_ref[...])

  return pl.pallas_call(
      body,
      grid=(n // bn,),
      in_specs=[
          pl.BlockSpec((bn, d), lambda i: (i, 0)),
          pl.BlockSpec((bn, d), lambda i: (i, 0)),
          pl.BlockSpec((bn, d), lambda i: (i, 0)),
          pl.BlockSpec((bn, d), lambda i: (i, 0)),
          pl.BlockSpec((d, d), lambda i: (0, 0)),
          pl.BlockSpec((d, d), lambda i: (0, 0)),
          pl.BlockSpec((1, d), lambda i: (0, 0)),
          pl.BlockSpec((d, co), lambda i: (0, 0)),
          pl.BlockSpec((1, co), lambda i: (0, 0)),
      ],
      out_specs=[pl.BlockSpec((bn, co), lambda i: (i, 0)),
                 pl.BlockSpec((bn, d), lambda i: (i, 0))],
      out_shape=[jax.ShapeDtypeStruct((n, co), jnp.float32),
                 jax.ShapeDtypeStruct((n, d), jnp.float32)],
  )(h, q0, q1, dinv, w_self, w_neigh, b.reshape(1, d), w_fc,
    b_fc.reshape(1, co))


def kernel(x, edge_index, W_self1, W_neigh1, b1, W_self2, W_neigh2, b2,
           W_fc, b_fc):
  n, d = x.shape
  e = edge_index.shape[1]
  nwk = _NC * _NS
  n2 = ((n + _NS * 8 - 1) // (_NS * 8)) * (_NS * 8)
  if n2 == n:
    n2 += _NS * 8

  assert e % nwk == 0
  ewr = e // nwk
  nch = -(-ewr // _CH)
  nch += nch % 2
  ndum = nch * _CH - ewr
  dum_dst = jnp.broadcast_to(n + jnp.arange(ndum, dtype=jnp.int32) % (n2 - n),
                             (nwk, ndum))
  src = jnp.concatenate(
      [edge_index[0].astype(jnp.int32).reshape(nwk, ewr),
       jnp.full((nwk, ndum), n, jnp.int32)], axis=1).reshape(nwk, nch, _CH)
  dst = jnp.concatenate(
      [edge_index[1].astype(jnp.int32).reshape(nwk, ewr),
       dum_dst], axis=1).reshape(nwk, nch, _CH)

  xpad = jnp.concatenate(
      [x, jnp.zeros((n2 - n, d), jnp.float32)], axis=0)
  zeros2 = jnp.zeros((n2, d), jnp.float32)
  zeros1 = jnp.zeros((n2,), jnp.float32)

  p, deg = _make_sc_agg(n2, nch, True)(xpad, src, dst, zeros2, zeros1)
  degt = deg.reshape(_NC, n2).T
  h, dinv = _tc_layer1(xpad, p[:n2], p[n2:], degt, W_self1, W_neigh1, b1, n)
  (q,) = _make_sc_agg(n2, nch, False)(h, src, dst, zeros2, zeros1)
  logits, h2 = _tc_layer2(h, q[:n2], q[n2:], dinv, W_self2, W_neigh2, b2,
                          W_fc, b_fc)
  return (logits[:n], h2[:n])


# speedup vs baseline: 2.6748x; 2.3704x over previous
import functools

import jax
import jax.numpy as jnp
from jax import lax
from jax.experimental import pallas as pl
from jax.experimental.pallas import tpu as pltpu
from jax.experimental.pallas import tpu_sc as plsc

_NC = 2
_NS = 16
_CH = 80


@functools.lru_cache(maxsize=None)
def _make_sc_agg(N, NCH, D, with_deg):
  ch = _CH
  rt = N // _NS
  assert N % (_NS * 8) == 0 and NCH % 2 == 1

  mesh = plsc.VectorSubcoreMesh(
      core_axis_name="c", subcore_axis_name="s",
      num_cores=_NC, num_subcores=_NS)

  out_type = [jax.ShapeDtypeStruct((_NC * N, D), jnp.float32)]
  scratch = [
      [pltpu.VMEM((ch,), jnp.int32) for _ in range(2)],
      [pltpu.VMEM((ch,), jnp.int32) for _ in range(2)],
      [pltpu.VMEM((ch, D), jnp.float32) for _ in range(2)],
      [pltpu.SemaphoreType.DMA for _ in range(2)],
      [pltpu.SemaphoreType.DMA for _ in range(2)],
      pltpu.VMEM_SHARED((N, D), jnp.float32),
  ]
  if with_deg:
    out_type.append(jax.ShapeDtypeStruct((_NC * N,), jnp.float32))
    scratch.append(pltpu.VMEM_SHARED((N,), jnp.float32))
    scratch.append(pltpu.VMEM((ch,), jnp.float32))

  @functools.partial(
      pl.kernel,
      mesh=mesh,
      compiler_params=pltpu.CompilerParams(use_tc_tiling_on_sc=False),
      out_type=out_type,
      scratch_types=scratch,
  )
  def sc_agg(feat_hbm, src_hbm, dst_hbm, zero2_hbm, zero1_hbm, *refs):
    if with_deg:
      (out_hbm, deg_hbm, sbuf, dbuf, rows, isems, gsems, acc, dacc,
       ones_v) = refs
    else:
      out_hbm, sbuf, dbuf, rows, isems, gsems, acc = refs

    c = lax.axis_index("c")
    s = lax.axis_index("s")
    r0 = s * rt
    pltpu.sync_copy(zero2_hbm.at[pl.ds(r0, rt)], acc.at[pl.ds(r0, rt)])
    if with_deg:
      pltpu.sync_copy(zero1_hbm.at[pl.ds(r0, rt)], dacc.at[pl.ds(r0, rt)])
      for i in range(ch // 16):
        ones_v[pl.ds(i * 16, 16)] = jnp.ones((16,), jnp.float32)
    plsc.subcore_barrier()

    wid = c * _NS + s

    def idx_copies(k, b):
      return (pltpu.make_async_copy(src_hbm.at[wid, k], sbuf[b], isems[b]),
              pltpu.make_async_copy(dst_hbm.at[wid, k], dbuf[b], isems[b]))

    def gather_copy(b):
      return pltpu.make_async_copy(feat_hbm.at[sbuf[b]], rows[b], gsems[b])

    for cp in idx_copies(0, 0):
      cp.start()
      cp.wait()
    for cp in idx_copies(1, 1):
      cp.start()
    gather_copy(0).start()

    def turn(k, b):
      k = jnp.int32(k)
      @pl.when(k + 1 < NCH)
      def _():
        for cp in idx_copies(k + 1, 1 - b):
          cp.wait()
        gather_copy(1 - b).start()

      gather_copy(b).wait()
      if with_deg:
        pltpu.sync_copy(ones_v, dacc.at[dbuf[b]], add=True)
      pltpu.sync_copy(rows[b], acc.at[dbuf[b]], add=True)

      @pl.when(k + 2 < NCH)
      def _():
        for cp in idx_copies(k + 2, b):
          cp.start()

    def body(o, carry):
      turn(2 * o, 0)
      turn(2 * o + 1, 1)
      return carry

    lax.fori_loop(0, NCH // 2, body, 0)
    turn(NCH - 1, 0)
    plsc.subcore_barrier()
    pltpu.sync_copy(acc.at[pl.ds(r0, rt)],
                    out_hbm.at[pl.ds(c * N + r0, rt)])
    if with_deg:
      pltpu.sync_copy(dacc.at[pl.ds(r0, rt)],
                      deg_hbm.at[pl.ds(c * N + r0, rt)])

  return sc_agg


def _tc_layer1(x, p0, p1, degt, w_self, w_neigh, b, n_real):
  n, d = x.shape
  bn = 1264
  assert n % bn == 0

  def body(x_ref, p0_ref, p1_ref, dg_ref, ws_ref, wn_ref, b_ref,
           h_ref, dinv_ref):
    dg = dg_ref[...]
    dinv = 1.0 / jnp.maximum(dg[:, 0:1] + dg[:, 1:2], 1.0)
    agg = (p0_ref[...] + p1_ref[...]) * dinv
    h = (jnp.dot(x_ref[...], ws_ref[...], preferred_element_type=jnp.float32)
         + jnp.dot(agg, wn_ref[...], preferred_element_type=jnp.float32)
         + b_ref[...])
    rowid = pl.program_id(0) * bn + lax.broadcasted_iota(jnp.int32, (bn, 1), 0)
    h_ref[...] = jnp.where(rowid < n_real, jnp.maximum(h, 0.0), 0.0)
    dinv_ref[...] = jnp.broadcast_to(dinv, (bn, d))

  return pl.pallas_call(
      body,
      grid=(n // bn,),
      in_specs=[
          pl.BlockSpec((bn, d), lambda i: (i, 0)),
          pl.BlockSpec((bn, d), lambda i: (i, 0)),
          pl.BlockSpec((bn, d), lambda i: (i, 0)),
          pl.BlockSpec((bn, 2), lambda i: (i, 0)),
          pl.BlockSpec((d, d), lambda i: (0, 0)),
          pl.BlockSpec((d, d), lambda i: (0, 0)),
          pl.BlockSpec((1, d), lambda i: (0, 0)),
      ],
      out_specs=[pl.BlockSpec((bn, d), lambda i: (i, 0)),
                 pl.BlockSpec((bn, d), lambda i: (i, 0))],
      out_shape=[jax.ShapeDtypeStruct((n, d), jnp.float32),
                 jax.ShapeDtypeStruct((n, d), jnp.float32)],
  )(x, p0, p1, degt, w_self, w_neigh, b.reshape(1, d))


def _tc_layer2(h, q0, q1, dinv, w_self, w_neigh, b, w_fc, b_fc):
  n, d = h.shape
  co = w_fc.shape[1]
  bn = 1264
  assert n % bn == 0

  def body(h_ref, q0_ref, q1_ref, dinv_ref, ws_ref, wn_ref, b_ref,
           wfc_ref, bfc_ref, logits_ref, h2_ref):
    agg = (q0_ref[...] + q1_ref[...]) * dinv_ref[...]
    h2 = (jnp.dot(h_ref[...], ws_ref[...], preferred_element_type=jnp.float32)
          + jnp.dot(agg, wn_ref[...], preferred_element_type=jnp.float32)
          + b_ref[...])
    h2_ref[...] = h2
    logits_ref[...] = (
        jnp.dot(h2, wfc_ref[...], preferred_element_type=jnp.float32)
        + bfc_ref[...])

  return pl.pallas_call(
      body,
      grid=(n // bn,),
      in_specs=[
          pl.BlockSpec((bn, d), lambda i: (i, 0)),
          pl.BlockSpec((bn, d), lambda i: (i, 0)),
          pl.BlockSpec((bn, d), lambda i: (i, 0)),
          pl.BlockSpec((bn, d), lambda i: (i, 0)),
          pl.BlockSpec((d, d), lambda i: (0, 0)),
          pl.BlockSpec((d, d), lambda i: (0, 0)),
          pl.BlockSpec((1, d), lambda i: (0, 0)),
          pl.BlockSpec((d, co), lambda i: (0, 0)),
          pl.BlockSpec((1, co), lambda i: (0, 0)),
      ],
      out_specs=[pl.BlockSpec((bn, co), lambda i: (i, 0)),
                 pl.BlockSpec((bn, d), lambda i: (i, 0))],
      out_shape=[jax.ShapeDtypeStruct((n, co), jnp.float32),
                 jax.ShapeDtypeStruct((n, d), jnp.float32)],
  )(h, q0, q1, dinv, w_self, w_neigh, b.reshape(1, d), w_fc,
    b_fc.reshape(1, co))


def kernel(x, edge_index, W_self1, W_neigh1, b1, W_self2, W_neigh2, b2,
           W_fc, b_fc):
  n, d = x.shape
  e = edge_index.shape[1]
  nwk = _NC * _NS
  n2 = ((n + _NS * 8 - 1) // (_NS * 8)) * (_NS * 8)
  if n2 == n:
    n2 += _NS * 8

  assert e % (nwk * _CH) == 0
  nch = e // (nwk * _CH)
  src = edge_index[0].astype(jnp.int32).reshape(nwk, nch, _CH)
  dst = edge_index[1].astype(jnp.int32).reshape(nwk, nch, _CH)

  xpad = jnp.concatenate(
      [x, jnp.zeros((n2 - n, d), jnp.float32)], axis=0)
  zeros2 = jnp.zeros((n2, d), jnp.float32)
  zeros1 = jnp.zeros((n2,), jnp.float32)

  p, deg = _make_sc_agg(n2, nch, d, True)(xpad, src, dst, zeros2, zeros1)
  degt = deg.reshape(_NC, n2).T
  h, dinv = _tc_layer1(xpad, p[:n2], p[n2:], degt, W_self1, W_neigh1, b1, n)
  (q,) = _make_sc_agg(n2, nch, d, False)(h, src, dst, zeros2, zeros1)
  logits, h2 = _tc_layer2(h, q[:n2], q[n2:], dinv, W_self2, W_neigh2, b2,
                          W_fc, b_fc)
  return (logits[:n], h2[:n])


# speedup vs baseline: 2.8260x; 1.0565x over previous
import functools

import jax
import jax.numpy as jnp
from jax import lax
from jax.experimental import pallas as pl
from jax.experimental.pallas import tpu as pltpu
from jax.experimental.pallas import tpu_sc as plsc

_NC = 2
_NS = 16
_CH = 80
_RD = 3


@functools.lru_cache(maxsize=None)
def _make_sc_agg(N, NCH, D, with_deg):
  ch = _CH
  rd = _RD
  rt = N // _NS
  assert N % (_NS * 8) == 0

  mesh = plsc.VectorSubcoreMesh(
      core_axis_name="c", subcore_axis_name="s",
      num_cores=_NC, num_subcores=_NS)

  out_type = [jax.ShapeDtypeStruct((_NC * N, D), jnp.float32)]
  scratch = [
      [pltpu.VMEM((ch,), jnp.int32) for _ in range(rd)],
      [pltpu.VMEM((ch,), jnp.int32) for _ in range(rd)],
      [pltpu.VMEM((ch, D), jnp.float32) for _ in range(rd)],
      [pltpu.SemaphoreType.DMA for _ in range(rd)],
      [pltpu.SemaphoreType.DMA for _ in range(rd)],
      pltpu.VMEM_SHARED((N, D), jnp.float32),
  ]
  if with_deg:
    out_type.append(jax.ShapeDtypeStruct((_NC * N,), jnp.float32))
    scratch.append(pltpu.VMEM_SHARED((N,), jnp.float32))
    scratch.append(pltpu.VMEM((ch,), jnp.float32))

  @functools.partial(
      pl.kernel,
      mesh=mesh,
      compiler_params=pltpu.CompilerParams(use_tc_tiling_on_sc=False),
      out_type=out_type,
      scratch_types=scratch,
  )
  def sc_agg(feat_hbm, src_hbm, dst_hbm, zero2_hbm, zero1_hbm, *refs):
    if with_deg:
      (out_hbm, deg_hbm, sbuf, dbuf, rows, isems, gsems, acc, dacc,
       ones_v) = refs
    else:
      out_hbm, sbuf, dbuf, rows, isems, gsems, acc = refs

    c = lax.axis_index("c")
    s = lax.axis_index("s")
    r0 = s * rt
    pltpu.sync_copy(zero2_hbm.at[pl.ds(r0, rt)], acc.at[pl.ds(r0, rt)])
    if with_deg:
      pltpu.sync_copy(zero1_hbm.at[pl.ds(r0, rt)], dacc.at[pl.ds(r0, rt)])
      for i in range(ch // 16):
        ones_v[pl.ds(i * 16, 16)] = jnp.ones((16,), jnp.float32)
    plsc.subcore_barrier()

    wid = c * _NS + s

    def idx_copies(k, b):
      return (pltpu.make_async_copy(src_hbm.at[wid, k], sbuf[b], isems[b]),
              pltpu.make_async_copy(dst_hbm.at[wid, k], dbuf[b], isems[b]))

    def gather_copy(b):
      return pltpu.make_async_copy(feat_hbm.at[sbuf[b]], rows[b], gsems[b])

    for j in range(rd):
      for cp in idx_copies(j, j):
        cp.start()
    for j in range(rd - 1):
      for cp in idx_copies(j, j):
        cp.wait()
      gather_copy(j).start()

    def turn(k, b):
      k = jnp.int32(k)
      bg = (b + rd - 1) % rd
      @pl.when(k + rd - 1 < NCH)
      def _():
        for cp in idx_copies(k + rd - 1, bg):
          cp.wait()
        gather_copy(bg).start()

      gather_copy(b).wait()
      if with_deg:
        pltpu.sync_copy(ones_v, dacc.at[dbuf[b]], add=True)
      pltpu.sync_copy(rows[b], acc.at[dbuf[b]], add=True)

      @pl.when(k + rd < NCH)
      def _():
        for cp in idx_copies(k + rd, b):
          cp.start()

    def body(o, carry):
      for j in range(rd):
        turn(rd * o + j, j)
      return carry

    lax.fori_loop(0, NCH // rd, body, 0)
    for k in range(NCH - NCH % rd, NCH):
      turn(k, k % rd)
    plsc.subcore_barrier()
    pltpu.sync_copy(acc.at[pl.ds(r0, rt)],
                    out_hbm.at[pl.ds(c * N + r0, rt)])
    if with_deg:
      pltpu.sync_copy(dacc.at[pl.ds(r0, rt)],
                      deg_hbm.at[pl.ds(c * N + r0, rt)])

  return sc_agg


def _tc_layer1(xpad, p, degt, w_self, w_neigh, b):
  n2, d = xpad.shape
  bn = n2 // 16
  nb = n2 // bn

  def body(x_ref, p0_ref, p1_ref, dg_ref, ws_ref, wn_ref, b_ref, h_ref):
    dg = dg_ref[...]
    dinv = 1.0 / jnp.maximum(dg[:, 0:1] + dg[:, 1:2], 1.0)
    agg = (p0_ref[...] + p1_ref[...]) * dinv
    h = (jnp.dot(x_ref[...], ws_ref[...], preferred_element_type=jnp.float32)
         + jnp.dot(agg, wn_ref[...], preferred_element_type=jnp.float32)
         + b_ref[...])
    h_ref[...] = jnp.maximum(h, 0.0)

  return pl.pallas_call(
      body,
      grid=(nb,),
      in_specs=[
          pl.BlockSpec((bn, d), lambda i: (i, 0)),
          pl.BlockSpec((bn, d), lambda i: (i, 0)),
          pl.BlockSpec((bn, d), lambda i: (nb + i, 0)),
          pl.BlockSpec((bn, 2), lambda i: (i, 0)),
          pl.BlockSpec((d, d), lambda i: (0, 0)),
          pl.BlockSpec((d, d), lambda i: (0, 0)),
          pl.BlockSpec((1, d), lambda i: (0, 0)),
      ],
      out_specs=pl.BlockSpec((bn, d), lambda i: (i, 0)),
      out_shape=jax.ShapeDtypeStruct((n2, d), jnp.float32),
  )(xpad, p, p, degt, w_self, w_neigh, b.reshape(1, d))


def _tc_layer2(h, q, degt, w_self, w_neigh, b, w_fc, b_fc):
  n2, d = h.shape
  co = w_fc.shape[1]
  bn = n2 // 16
  nb = n2 // bn

  def body(h_ref, q0_ref, q1_ref, dg_ref, ws_ref, wn_ref, b_ref,
           wfc_ref, bfc_ref, logits_ref, h2_ref):
    dg = dg_ref[...]
    dinv = 1.0 / jnp.maximum(dg[:, 0:1] + dg[:, 1:2], 1.0)
    agg = (q0_ref[...] + q1_ref[...]) * dinv
    h2 = (jnp.dot(h_ref[...], ws_ref[...], preferred_element_type=jnp.float32)
          + jnp.dot(agg, wn_ref[...], preferred_element_type=jnp.float32)
          + b_ref[...])
    h2_ref[...] = h2
    logits_ref[...] = (
        jnp.dot(h2, wfc_ref[...], preferred_element_type=jnp.float32)
        + bfc_ref[...])

  return pl.pallas_call(
      body,
      grid=(nb,),
      in_specs=[
          pl.BlockSpec((bn, d), lambda i: (i, 0)),
          pl.BlockSpec((bn, d), lambda i: (i, 0)),
          pl.BlockSpec((bn, d), lambda i: (nb + i, 0)),
          pl.BlockSpec((bn, 2), lambda i: (i, 0)),
          pl.BlockSpec((d, d), lambda i: (0, 0)),
          pl.BlockSpec((d, d), lambda i: (0, 0)),
          pl.BlockSpec((1, d), lambda i: (0, 0)),
          pl.BlockSpec((d, co), lambda i: (0, 0)),
          pl.BlockSpec((1, co), lambda i: (0, 0)),
      ],
      out_specs=[pl.BlockSpec((bn, co), lambda i: (i, 0)),
                 pl.BlockSpec((bn, d), lambda i: (i, 0))],
      out_shape=[jax.ShapeDtypeStruct((n2, co), jnp.float32),
                 jax.ShapeDtypeStruct((n2, d), jnp.float32)],
  )(h, q, q, degt, w_self, w_neigh, b.reshape(1, d), w_fc,
    b_fc.reshape(1, co))


def kernel(x, edge_index, W_self1, W_neigh1, b1, W_self2, W_neigh2, b2,
           W_fc, b_fc):
  n, d = x.shape
  e = edge_index.shape[1]
  nwk = _NC * _NS
  n2 = ((n + _NS * 8 - 1) // (_NS * 8)) * (_NS * 8)

  assert e % (nwk * _CH) == 0
  nch = e // (nwk * _CH)
  src = edge_index[0].astype(jnp.int32).reshape(nwk, nch, _CH)
  dst = edge_index[1].astype(jnp.int32).reshape(nwk, nch, _CH)

  xpad = jnp.concatenate([x, jnp.zeros((n2 - n, d), jnp.float32)], axis=0)
  zeros2 = jnp.zeros((n2, d), jnp.float32)
  zeros1 = jnp.zeros((n2,), jnp.float32)

  p, deg = _make_sc_agg(n2, nch, d, True)(xpad, src, dst, zeros2, zeros1)
  degt = deg.reshape(_NC, n2).T
  h = _tc_layer1(xpad, p, degt, W_self1, W_neigh1, b1)
  (q,) = _make_sc_agg(n2, nch, d, False)(h, src, dst, zeros2, zeros1)
  logits, h2 = _tc_layer2(h, q, degt, W_self2, W_neigh2, b2, W_fc, b_fc)
  return (logits[:n], h2[:n])


# speedup vs baseline: 2.8611x; 1.0124x over previous
import functools

import jax
import jax.numpy as jnp
from jax import lax
from jax.experimental import pallas as pl
from jax.experimental.pallas import tpu as pltpu
from jax.experimental.pallas import tpu_sc as plsc

_NC = 2
_NS = 16
_CH = 80
_RD = 4


@functools.lru_cache(maxsize=None)
def _make_sc_agg(N, NCH, D, with_deg):
  ch = _CH
  rd = _RD
  rt = N // _NS
  assert N % (_NS * 8) == 0

  mesh = plsc.VectorSubcoreMesh(
      core_axis_name="c", subcore_axis_name="s",
      num_cores=_NC, num_subcores=_NS)

  out_type = [jax.ShapeDtypeStruct((_NC * N, D), jnp.float32)]
  scratch = [
      [pltpu.VMEM((ch,), jnp.int32) for _ in range(rd)],
      [pltpu.VMEM((ch,), jnp.int32) for _ in range(rd)],
      [pltpu.VMEM((ch, D), jnp.float32) for _ in range(rd)],
      [pltpu.SemaphoreType.DMA for _ in range(rd)],
      [pltpu.SemaphoreType.DMA for _ in range(rd)],
      pltpu.VMEM_SHARED((N, D), jnp.float32),
      pltpu.VMEM((ch,), jnp.float32),
  ]
  if with_deg:
    out_type.append(jax.ShapeDtypeStruct((_NC * N,), jnp.float32))
    scratch.append(pltpu.VMEM_SHARED((N,), jnp.float32))
    scratch.append(pltpu.VMEM((ch,), jnp.float32))

  @functools.partial(
      pl.kernel,
      mesh=mesh,
      compiler_params=pltpu.CompilerParams(use_tc_tiling_on_sc=False),
      out_type=out_type,
      scratch_types=scratch,
  )
  def sc_agg(feat_hbm, src_hbm, dst_hbm, *refs):
    if with_deg:
      (out_hbm, deg_hbm, sbuf, dbuf, rows, isems, gsems, acc, zero_v, dacc,
       ones_v) = refs
    else:
      out_hbm, sbuf, dbuf, rows, isems, gsems, acc, zero_v = refs

    c = lax.axis_index("c")
    s = lax.axis_index("s")
    r0 = s * rt
    def zloop(j, carry):
      rows[0][j // (D // 16), pl.ds((j % (D // 16)) * 16, 16)] = (
          jnp.zeros((16,), jnp.float32))
      return carry
    lax.fori_loop(0, ch * D // 16, zloop, 0)
    for i in range(ch // 16):
      zero_v[pl.ds(i * 16, 16)] = jnp.zeros((16,), jnp.float32)
      if with_deg:
        ones_v[pl.ds(i * 16, 16)] = jnp.ones((16,), jnp.float32)
    nz = rt // ch
    for j in range(nz):
      pltpu.sync_copy(rows[0].at[pl.ds(0, ch)],
                      acc.at[pl.ds(r0 + j * ch, ch)])
      if with_deg:
        pltpu.sync_copy(zero_v, dacc.at[pl.ds(r0 + j * ch, ch)])
    rem = rt - nz * ch
    if rem:
      pltpu.sync_copy(rows[0].at[pl.ds(0, rem)],
                      acc.at[pl.ds(r0 + nz * ch, rem)])
      if with_deg:
        pltpu.sync_copy(zero_v.at[pl.ds(0, rem)],
                        dacc.at[pl.ds(r0 + nz * ch, rem)])
    plsc.subcore_barrier()

    wid = c * _NS + s

    def idx_copies(k, b):
      return (pltpu.make_async_copy(src_hbm.at[wid, k], sbuf[b], isems[b]),
              pltpu.make_async_copy(dst_hbm.at[wid, k], dbuf[b], isems[b]))

    def gather_copy(b):
      return pltpu.make_async_copy(feat_hbm.at[sbuf[b]], rows[b], gsems[b])

    for j in range(rd):
      for cp in idx_copies(j, j):
        cp.start()
    for j in range(rd - 1):
      for cp in idx_copies(j, j):
        cp.wait()
      gather_copy(j).start()

    def turn(k, b):
      k = jnp.int32(k)
      bg = (b + rd - 1) % rd
      @pl.when(k + rd - 1 < NCH)
      def _():
        for cp in idx_copies(k + rd - 1, bg):
          cp.wait()
        gather_copy(bg).start()

      gather_copy(b).wait()
      if with_deg:
        pltpu.sync_copy(ones_v, dacc.at[dbuf[b]], add=True)
      pltpu.sync_copy(rows[b], acc.at[dbuf[b]], add=True)

      @pl.when(k + rd < NCH)
      def _():
        for cp in idx_copies(k + rd, b):
          cp.start()

    def body(o, carry):
      for j in range(rd):
        turn(rd * o + j, j)
      return carry

    lax.fori_loop(0, NCH // rd, body, 0)
    for k in range(NCH - NCH % rd, NCH):
      turn(k, k % rd)
    plsc.subcore_barrier()
    pltpu.sync_copy(acc.at[pl.ds(r0, rt)],
                    out_hbm.at[pl.ds(c * N + r0, rt)])
    if with_deg:
      pltpu.sync_copy(dacc.at[pl.ds(r0, rt)],
                      deg_hbm.at[pl.ds(c * N + r0, rt)])

  return sc_agg


def _tc_layer1(xpad, p, degt, w_self, w_neigh, b):
  n2, d = xpad.shape
  bn = n2 // 16
  nb = n2 // bn

  def body(x_ref, p0_ref, p1_ref, dg_ref, ws_ref, wn_ref, b_ref, h_ref):
    dg = dg_ref[...]
    dinv = 1.0 / jnp.maximum(dg[:, 0:1] + dg[:, 1:2], 1.0)
    agg = (p0_ref[...] + p1_ref[...]) * dinv
    h = (jnp.dot(x_ref[...], ws_ref[...], preferred_element_type=jnp.float32)
         + jnp.dot(agg, wn_ref[...], preferred_element_type=jnp.float32)
         + b_ref[...])
    h_ref[...] = jnp.maximum(h, 0.0)

  return pl.pallas_call(
      body,
      grid=(nb,),
      in_specs=[
          pl.BlockSpec((bn, d), lambda i: (i, 0)),
          pl.BlockSpec((bn, d), lambda i: (i, 0)),
          pl.BlockSpec((bn, d), lambda i: (nb + i, 0)),
          pl.BlockSpec((bn, 2), lambda i: (i, 0)),
          pl.BlockSpec((d, d), lambda i: (0, 0)),
          pl.BlockSpec((d, d), lambda i: (0, 0)),
          pl.BlockSpec((1, d), lambda i: (0, 0)),
      ],
      out_specs=pl.BlockSpec((bn, d), lambda i: (i, 0)),
      out_shape=jax.ShapeDtypeStruct((n2, d), jnp.float32),
  )(xpad, p, p, degt, w_self, w_neigh, b.reshape(1, d))


def _tc_layer2(h, q, degt, w_self, w_neigh, b, w_fc, b_fc):
  n2, d = h.shape
  co = w_fc.shape[1]
  bn = n2 // 16
  nb = n2 // bn

  def body(h_ref, q0_ref, q1_ref, dg_ref, ws_ref, wn_ref, b_ref,
           wfc_ref, bfc_ref, logits_ref, h2_ref):
    dg = dg_ref[...]
    dinv = 1.0 / jnp.maximum(dg[:, 0:1] + dg[:, 1:2], 1.0)
    agg = (q0_ref[...] + q1_ref[...]) * dinv
    h2 = (jnp.dot(h_ref[...], ws_ref[...], preferred_element_type=jnp.float32)
          + jnp.dot(agg, wn_ref[...], preferred_element_type=jnp.float32)
          + b_ref[...])
    h2_ref[...] = h2
    logits_ref[...] = (
        jnp.dot(h2, wfc_ref[...], preferred_element_type=jnp.float32)
        + bfc_ref[...])

  return pl.pallas_call(
      body,
      grid=(nb,),
      in_specs=[
          pl.BlockSpec((bn, d), lambda i: (i, 0)),
          pl.BlockSpec((bn, d), lambda i: (i, 0)),
          pl.BlockSpec((bn, d), lambda i: (nb + i, 0)),
          pl.BlockSpec((bn, 2), lambda i: (i, 0)),
          pl.BlockSpec((d, d), lambda i: (0, 0)),
          pl.BlockSpec((d, d), lambda i: (0, 0)),
          pl.BlockSpec((1, d), lambda i: (0, 0)),
          pl.BlockSpec((d, co), lambda i: (0, 0)),
          pl.BlockSpec((1, co), lambda i: (0, 0)),
      ],
      out_specs=[pl.BlockSpec((bn, co), lambda i: (i, 0)),
                 pl.BlockSpec((bn, d), lambda i: (i, 0))],
      out_shape=[jax.ShapeDtypeStruct((n2, co), jnp.float32),
                 jax.ShapeDtypeStruct((n2, d), jnp.float32)],
  )(h, q, q, degt, w_self, w_neigh, b.reshape(1, d), w_fc,
    b_fc.reshape(1, co))


def kernel(x, edge_index, W_self1, W_neigh1, b1, W_self2, W_neigh2, b2,
           W_fc, b_fc):
  n, d = x.shape
  e = edge_index.shape[1]
  nwk = _NC * _NS
  n2 = ((n + _NS * 8 - 1) // (_NS * 8)) * (_NS * 8)

  assert e % (nwk * _CH) == 0
  nch = e // (nwk * _CH)
  src = edge_index[0].astype(jnp.int32).reshape(nwk, nch, _CH)
  dst = edge_index[1].astype(jnp.int32).reshape(nwk, nch, _CH)

  xpad = jnp.concatenate([x, jnp.zeros((n2 - n, d), jnp.float32)], axis=0)

  p, deg = _make_sc_agg(n2, nch, d, True)(xpad, src, dst)
  degt = deg.reshape(_NC, n2).T
  h = _tc_layer1(xpad, p, degt, W_self1, W_neigh1, b1)
  (q,) = _make_sc_agg(n2, nch, d, False)(h, src, dst)
  logits, h2 = _tc_layer2(h, q, degt, W_self2, W_neigh2, b2, W_fc, b_fc)
  return (logits[:n], h2[:n])


# speedup vs baseline: 2.9195x; 1.0204x over previous
import functools

import jax
import jax.numpy as jnp
from jax import lax
from jax.experimental import pallas as pl
from jax.experimental.pallas import tpu as pltpu
from jax.experimental.pallas import tpu_sc as plsc

_NC = 2
_NS = 16
_CH = 80
_RD = 4


@functools.lru_cache(maxsize=None)
def _make_sc_agg(N, NF, NCH, D, with_deg):
  ch = _CH
  rd = _RD
  rt = N // _NS
  assert N % (_NS * 8) == 0

  mesh = plsc.VectorSubcoreMesh(
      core_axis_name="c", subcore_axis_name="s",
      num_cores=_NC, num_subcores=_NS)

  out_type = [jax.ShapeDtypeStruct((_NC * N, D), jnp.float32)]
  scratch = [
      [pltpu.VMEM((ch,), jnp.int32) for _ in range(rd)],
      [pltpu.VMEM((ch,), jnp.int32) for _ in range(rd)],
      [pltpu.VMEM((ch, D), jnp.float32) for _ in range(rd)],
      [pltpu.SemaphoreType.DMA for _ in range(rd)],
      [pltpu.SemaphoreType.DMA for _ in range(rd)],
      pltpu.VMEM_SHARED((N, D), jnp.float32),
      pltpu.VMEM((ch,), jnp.float32),
  ]
  if with_deg:
    out_type.append(jax.ShapeDtypeStruct((_NC * N,), jnp.float32))
    scratch.append(pltpu.VMEM_SHARED((N,), jnp.float32))
    scratch.append(pltpu.VMEM((ch,), jnp.float32))

  @functools.partial(
      pl.kernel,
      mesh=mesh,
      compiler_params=pltpu.CompilerParams(use_tc_tiling_on_sc=False),
      out_type=out_type,
      scratch_types=scratch,
  )
  def sc_agg(feat_hbm, src_hbm, dst_hbm, *refs):
    if with_deg:
      (out_hbm, deg_hbm, sbuf, dbuf, rows, isems, gsems, acc, zero_v, dacc,
       ones_v) = refs
    else:
      out_hbm, sbuf, dbuf, rows, isems, gsems, acc, zero_v = refs

    c = lax.axis_index("c")
    s = lax.axis_index("s")
    r0 = s * rt
    def zloop(j, carry):
      rows[0][j // (D // 16), pl.ds((j % (D // 16)) * 16, 16)] = (
          jnp.zeros((16,), jnp.float32))
      return carry
    lax.fori_loop(0, ch * D // 16, zloop, 0)
    for i in range(ch // 16):
      zero_v[pl.ds(i * 16, 16)] = jnp.zeros((16,), jnp.float32)
      if with_deg:
        ones_v[pl.ds(i * 16, 16)] = jnp.ones((16,), jnp.float32)
    nz = rt // ch
    for j in range(nz):
      pltpu.sync_copy(rows[0].at[pl.ds(0, ch)],
                      acc.at[pl.ds(r0 + j * ch, ch)])
      if with_deg:
        pltpu.sync_copy(zero_v, dacc.at[pl.ds(r0 + j * ch, ch)])
    rem = rt - nz * ch
    if rem:
      pltpu.sync_copy(rows[0].at[pl.ds(0, rem)],
                      acc.at[pl.ds(r0 + nz * ch, rem)])
      if with_deg:
        pltpu.sync_copy(zero_v.at[pl.ds(0, rem)],
                        dacc.at[pl.ds(r0 + nz * ch, rem)])
    plsc.subcore_barrier()

    wid = c * _NS + s

    def idx_copies(k, b):
      return (pltpu.make_async_copy(src_hbm.at[wid, k], sbuf[b], isems[b]),
              pltpu.make_async_copy(dst_hbm.at[wid, k], dbuf[b], isems[b]))

    def gather_copy(b):
      return pltpu.make_async_copy(feat_hbm.at[sbuf[b]], rows[b], gsems[b])

    for j in range(rd):
      for cp in idx_copies(j, j):
        cp.start()
    for j in range(rd - 1):
      for cp in idx_copies(j, j):
        cp.wait()
      gather_copy(j).start()

    def turn(k, b):
      k = jnp.int32(k)
      bg = (b + rd - 1) % rd
      @pl.when(k + rd - 1 < NCH)
      def _():
        for cp in idx_copies(k + rd - 1, bg):
          cp.wait()
        gather_copy(bg).start()

      gather_copy(b).wait()
      if with_deg:
        pltpu.sync_copy(ones_v, dacc.at[dbuf[b]], add=True)
      pltpu.sync_copy(rows[b], acc.at[dbuf[b]], add=True)

      @pl.when(k + rd < NCH)
      def _():
        for cp in idx_copies(k + rd, b):
          cp.start()

    def body(o, carry):
      for j in range(rd):
        turn(rd * o + j, j)
      return carry

    lax.fori_loop(0, NCH // rd, body, 0)
    for k in range(NCH - NCH % rd, NCH):
      turn(k, k % rd)
    plsc.subcore_barrier()
    pltpu.sync_copy(acc.at[pl.ds(r0, rt)],
                    out_hbm.at[pl.ds(c * N + r0, rt)])
    if with_deg:
      pltpu.sync_copy(dacc.at[pl.ds(r0, rt)],
                      deg_hbm.at[pl.ds(c * N + r0, rt)])

  return sc_agg


def _tc_layer1(x, p3, deg3, w_self, w_neigh, b):
  n, d = x.shape
  bn = 1000
  assert n % bn == 0

  def body(x_ref, p0_ref, p1_ref, d0_ref, d1_ref, ws_ref, wn_ref, b_ref,
           h_ref):
    dg = d0_ref[0] + d1_ref[0]
    dinv = 1.0 / jnp.maximum(dg, 1.0)
    agg = (p0_ref[0] + p1_ref[0]) * dinv
    h = (jnp.dot(x_ref[...], ws_ref[...], preferred_element_type=jnp.float32)
         + jnp.dot(agg, wn_ref[...], preferred_element_type=jnp.float32)
         + b_ref[...])
    h_ref[...] = jnp.maximum(h, 0.0)

  return pl.pallas_call(
      body,
      grid=(n // bn,),
      in_specs=[
          pl.BlockSpec((bn, d), lambda i: (i, 0)),
          pl.BlockSpec((1, bn, d), lambda i: (0, i, 0)),
          pl.BlockSpec((1, bn, d), lambda i: (1, i, 0)),
          pl.BlockSpec((1, bn, 1), lambda i: (0, i, 0)),
          pl.BlockSpec((1, bn, 1), lambda i: (1, i, 0)),
          pl.BlockSpec((d, d), lambda i: (0, 0)),
          pl.BlockSpec((d, d), lambda i: (0, 0)),
          pl.BlockSpec((1, d), lambda i: (0, 0)),
      ],
      out_specs=pl.BlockSpec((bn, d), lambda i: (i, 0)),
      out_shape=jax.ShapeDtypeStruct((n, d), jnp.float32),
  )(x, p3, p3, deg3, deg3, w_self, w_neigh, b.reshape(1, d))


def _tc_layer2(h, q3, deg3, w_self, w_neigh, b, w_fc, b_fc):
  n, d = h.shape
  co = w_fc.shape[1]
  bn = 1000
  assert n % bn == 0

  def body(h_ref, q0_ref, q1_ref, d0_ref, d1_ref, ws_ref, wn_ref, b_ref,
           wfc_ref, bfc_ref, logits_ref, h2_ref):
    dg = d0_ref[0] + d1_ref[0]
    dinv = 1.0 / jnp.maximum(dg, 1.0)
    agg = (q0_ref[0] + q1_ref[0]) * dinv
    h2 = (jnp.dot(h_ref[...], ws_ref[...], preferred_element_type=jnp.float32)
          + jnp.dot(agg, wn_ref[...], preferred_element_type=jnp.float32)
          + b_ref[...])
    h2_ref[...] = h2
    logits_ref[...] = (
        jnp.dot(h2, wfc_ref[...], preferred_element_type=jnp.float32)
        + bfc_ref[...])

  return pl.pallas_call(
      body,
      grid=(n // bn,),
      in_specs=[
          pl.BlockSpec((bn, d), lambda i: (i, 0)),
          pl.BlockSpec((1, bn, d), lambda i: (0, i, 0)),
          pl.BlockSpec((1, bn, d), lambda i: (1, i, 0)),
          pl.BlockSpec((1, bn, 1), lambda i: (0, i, 0)),
          pl.BlockSpec((1, bn, 1), lambda i: (1, i, 0)),
          pl.BlockSpec((d, d), lambda i: (0, 0)),
          pl.BlockSpec((d, d), lambda i: (0, 0)),
          pl.BlockSpec((1, d), lambda i: (0, 0)),
          pl.BlockSpec((d, co), lambda i: (0, 0)),
          pl.BlockSpec((1, co), lambda i: (0, 0)),
      ],
      out_specs=[pl.BlockSpec((bn, co), lambda i: (i, 0)),
                 pl.BlockSpec((bn, d), lambda i: (i, 0))],
      out_shape=[jax.ShapeDtypeStruct((n, co), jnp.float32),
                 jax.ShapeDtypeStruct((n, d), jnp.float32)],
  )(h, q3, q3, deg3, deg3, w_self, w_neigh, b.reshape(1, d), w_fc,
    b_fc.reshape(1, co))


def kernel(x, edge_index, W_self1, W_neigh1, b1, W_self2, W_neigh2, b2,
           W_fc, b_fc):
  n, d = x.shape
  e = edge_index.shape[1]
  nwk = _NC * _NS
  n2 = ((n + _NS * 8 - 1) // (_NS * 8)) * (_NS * 8)

  assert e % (nwk * _CH) == 0
  nch = e // (nwk * _CH)
  src = edge_index[0].astype(jnp.int32).reshape(nwk, nch, _CH)
  dst = edge_index[1].astype(jnp.int32).reshape(nwk, nch, _CH)

  p, deg = _make_sc_agg(n2, n, nch, d, True)(x, src, dst)
  p3 = p.reshape(_NC, n2, d)
  deg3 = deg.reshape(_NC, n2, 1)
  h = _tc_layer1(x, p3, deg3, W_self1, W_neigh1, b1)
  (q,) = _make_sc_agg(n2, n, nch, d, False)(h, src, dst)
  q3 = q.reshape(_NC, n2, d)
  logits, h2 = _tc_layer2(h, q3, deg3, W_self2, W_neigh2, b2, W_fc, b_fc)
  return (logits, h2)


# speedup vs baseline: 3.6406x; 1.2470x over previous
import functools

import jax
import jax.numpy as jnp
from jax import lax
from jax.experimental import pallas as pl
from jax.experimental.pallas import tpu as pltpu
from jax.experimental.pallas import tpu_sc as plsc

_NC = 2
_NS = 16
_CH = 80
_RD = 4


@functools.lru_cache(maxsize=None)
def _make_sc_agg(N, NF, NCH, D, with_deg):
  ch = _CH
  rd = 4
  ri = 2 * rd
  rt = N // _NS
  assert N % (_NS * 8) == 0

  mesh = plsc.VectorSubcoreMesh(
      core_axis_name="c", subcore_axis_name="s",
      num_cores=_NC, num_subcores=_NS)

  out_type = [jax.ShapeDtypeStruct((_NC * N, D), jnp.float32)]
  scratch = [
      [pltpu.VMEM((ch,), jnp.int32) for _ in range(ri)],
      [pltpu.VMEM((ch,), jnp.int32) for _ in range(ri)],
      [pltpu.VMEM((ch, D), jnp.float32) for _ in range(rd)],
      [pltpu.SemaphoreType.DMA for _ in range(ri)],
      [pltpu.SemaphoreType.DMA for _ in range(rd)],
      [pltpu.SemaphoreType.DMA for _ in range(rd)],
      pltpu.VMEM_SHARED((N, D), jnp.float32),
      pltpu.VMEM((ch,), jnp.float32),
  ]
  if with_deg:
    out_type.append(jax.ShapeDtypeStruct((_NC * N,), jnp.float32))
    scratch.append(pltpu.VMEM_SHARED((N,), jnp.float32))
    scratch.append(pltpu.VMEM((ch,), jnp.float32))

  @functools.partial(
      pl.kernel,
      mesh=mesh,
      compiler_params=pltpu.CompilerParams(use_tc_tiling_on_sc=False),
      out_type=out_type,
      scratch_types=scratch,
  )
  def sc_agg(feat_hbm, src_hbm, dst_hbm, *refs):
    if with_deg:
      (out_hbm, deg_hbm, sbuf, dbuf, rows, isems, gsems, ssems, acc, zero_v,
       dacc, ones_v) = refs
    else:
      out_hbm, sbuf, dbuf, rows, isems, gsems, ssems, acc, zero_v = refs

    c = lax.axis_index("c")
    s = lax.axis_index("s")
    r0 = s * rt
    def zloop(j, carry):
      rows[0][j // (D // 16), pl.ds((j % (D // 16)) * 16, 16)] = (
          jnp.zeros((16,), jnp.float32))
      return carry
    lax.fori_loop(0, ch * D // 16, zloop, 0)
    for i in range(ch // 16):
      zero_v[pl.ds(i * 16, 16)] = jnp.zeros((16,), jnp.float32)
      if with_deg:
        ones_v[pl.ds(i * 16, 16)] = jnp.ones((16,), jnp.float32)
    nz = rt // ch
    for j in range(nz):
      pltpu.sync_copy(rows[0].at[pl.ds(0, ch)],
                      acc.at[pl.ds(r0 + j * ch, ch)])
      if with_deg:
        pltpu.sync_copy(zero_v, dacc.at[pl.ds(r0 + j * ch, ch)])
    rem = rt - nz * ch
    if rem:
      pltpu.sync_copy(rows[0].at[pl.ds(0, rem)],
                      acc.at[pl.ds(r0 + nz * ch, rem)])
      if with_deg:
        pltpu.sync_copy(zero_v.at[pl.ds(0, rem)],
                        dacc.at[pl.ds(r0 + nz * ch, rem)])
    plsc.subcore_barrier()

    wid = c * _NS + s

    def idx_copies(k, bi):
      return (pltpu.make_async_copy(src_hbm.at[wid, k], sbuf[bi], isems[bi]),
              pltpu.make_async_copy(dst_hbm.at[wid, k], dbuf[bi], isems[bi]))

    def gather_copy(br, bi):
      return pltpu.make_async_copy(feat_hbm.at[sbuf[bi]], rows[br], gsems[br])

    def scatter_start(br, bi):
      if with_deg:
        pltpu.async_copy(ones_v, dacc.at[dbuf[bi]], ssems[br], add=True)
      pltpu.async_copy(rows[br], acc.at[dbuf[bi]], ssems[br], add=True)

    def scatter_wait(br, bi):
      if with_deg:
        pltpu.make_async_copy(ones_v, dacc.at[dbuf[bi]], ssems[br]).wait()
      pltpu.make_async_copy(rows[br], acc.at[dbuf[bi]], ssems[br]).wait()

    for j in range(ri - 2):
      for cp in idx_copies(j, j):
        cp.start()
    for j in range(2):
      for cp in idx_copies(j, j):
        cp.wait()
      gather_copy(j, j).start()

    def turn(k, j):
      k = jnp.int32(k)
      @pl.when(k >= 2)
      def _():
        scatter_wait((j - 2) % rd, (j - 2) % ri)

      @pl.when(k + ri - 2 < NCH)
      def _():
        for cp in idx_copies(k + ri - 2, (j + ri - 2) % ri):
          cp.start()

      @pl.when(k + 2 < NCH)
      def _():
        for cp in idx_copies(k + 2, (j + 2) % ri):
          cp.wait()
        gather_copy((j + 2) % rd, (j + 2) % ri).start()

      gather_copy(j % rd, j % ri).wait()
      scatter_start(j % rd, j % ri)

    def body(o, carry):
      for j in range(ri):
        turn(ri * o + j, j)
      return carry

    lax.fori_loop(0, NCH // ri, body, 0)
    for k in range(NCH - NCH % ri, NCH):
      turn(k, k % ri)
    for k in (NCH - 2, NCH - 1):
      scatter_wait(k % rd, k % ri)
    plsc.subcore_barrier()
    pltpu.sync_copy(acc.at[pl.ds(r0, rt)],
                    out_hbm.at[pl.ds(c * N + r0, rt)])
    if with_deg:
      pltpu.sync_copy(dacc.at[pl.ds(r0, rt)],
                      deg_hbm.at[pl.ds(c * N + r0, rt)])

  return sc_agg


def _tc_layer1(x, p3, deg3, w_self, w_neigh, b):
  n, d = x.shape
  bn = 1000
  assert n % bn == 0

  def body(x_ref, p0_ref, p1_ref, d0_ref, d1_ref, ws_ref, wn_ref, b_ref,
           h_ref):
    dg = d0_ref[0] + d1_ref[0]
    dinv = 1.0 / jnp.maximum(dg, 1.0)
    agg = (p0_ref[0] + p1_ref[0]) * dinv
    h = (jnp.dot(x_ref[...], ws_ref[...], preferred_element_type=jnp.float32)
         + jnp.dot(agg, wn_ref[...], preferred_element_type=jnp.float32)
         + b_ref[...])
    h_ref[...] = jnp.maximum(h, 0.0)

  return pl.pallas_call(
      body,
      grid=(n // bn,),
      in_specs=[
          pl.BlockSpec((bn, d), lambda i: (i, 0)),
          pl.BlockSpec((1, bn, d), lambda i: (0, i, 0)),
          pl.BlockSpec((1, bn, d), lambda i: (1, i, 0)),
          pl.BlockSpec((1, bn, 1), lambda i: (0, i, 0)),
          pl.BlockSpec((1, bn, 1), lambda i: (1, i, 0)),
          pl.BlockSpec((d, d), lambda i: (0, 0)),
          pl.BlockSpec((d, d), lambda i: (0, 0)),
          pl.BlockSpec((1, d), lambda i: (0, 0)),
      ],
      out_specs=pl.BlockSpec((bn, d), lambda i: (i, 0)),
      out_shape=jax.ShapeDtypeStruct((n, d), jnp.float32),
  )(x, p3, p3, deg3, deg3, w_self, w_neigh, b.reshape(1, d))


def _tc_layer2(h, q3, deg3, w_self, w_neigh, b, w_fc, b_fc):
  n, d = h.shape
  co = w_fc.shape[1]
  bn = 1000
  assert n % bn == 0

  def body(h_ref, q0_ref, q1_ref, d0_ref, d1_ref, ws_ref, wn_ref, b_ref,
           wfc_ref, bfc_ref, logits_ref, h2_ref):
    dg = d0_ref[0] + d1_ref[0]
    dinv = 1.0 / jnp.maximum(dg, 1.0)
    agg = (q0_ref[0] + q1_ref[0]) * dinv
    h2 = (jnp.dot(h_ref[...], ws_ref[...], preferred_element_type=jnp.float32)
          + jnp.dot(agg, wn_ref[...], preferred_element_type=jnp.float32)
          + b_ref[...])
    h2_ref[...] = h2
    logits_ref[...] = (
        jnp.dot(h2, wfc_ref[...], preferred_element_type=jnp.float32)
        + bfc_ref[...])

  return pl.pallas_call(
      body,
      grid=(n // bn,),
      in_specs=[
          pl.BlockSpec((bn, d), lambda i: (i, 0)),
          pl.BlockSpec((1, bn, d), lambda i: (0, i, 0)),
          pl.BlockSpec((1, bn, d), lambda i: (1, i, 0)),
          pl.BlockSpec((1, bn, 1), lambda i: (0, i, 0)),
          pl.BlockSpec((1, bn, 1), lambda i: (1, i, 0)),
          pl.BlockSpec((d, d), lambda i: (0, 0)),
          pl.BlockSpec((d, d), lambda i: (0, 0)),
          pl.BlockSpec((1, d), lambda i: (0, 0)),
          pl.BlockSpec((d, co), lambda i: (0, 0)),
          pl.BlockSpec((1, co), lambda i: (0, 0)),
      ],
      out_specs=[pl.BlockSpec((bn, co), lambda i: (i, 0)),
                 pl.BlockSpec((bn, d), lambda i: (i, 0))],
      out_shape=[jax.ShapeDtypeStruct((n, co), jnp.float32),
                 jax.ShapeDtypeStruct((n, d), jnp.float32)],
  )(h, q3, q3, deg3, deg3, w_self, w_neigh, b.reshape(1, d), w_fc,
    b_fc.reshape(1, co))


def kernel(x, edge_index, W_self1, W_neigh1, b1, W_self2, W_neigh2, b2,
           W_fc, b_fc):
  n, d = x.shape
  e = edge_index.shape[1]
  nwk = _NC * _NS
  n2 = ((n + _NS * 8 - 1) // (_NS * 8)) * (_NS * 8)

  assert e % (nwk * _CH) == 0
  nch = e // (nwk * _CH)
  src = edge_index[0].astype(jnp.int32).reshape(nwk, nch, _CH)
  dst = edge_index[1].astype(jnp.int32).reshape(nwk, nch, _CH)

  p, deg = _make_sc_agg(n2, n, nch, d, True)(x, src, dst)
  p3 = p.reshape(_NC, n2, d)
  deg3 = deg.reshape(_NC, n2, 1)
  h = _tc_layer1(x, p3, deg3, W_self1, W_neigh1, b1)
  (q,) = _make_sc_agg(n2, n, nch, d, False)(h, src, dst)
  q3 = q.reshape(_NC, n2, d)
  logits, h2 = _tc_layer2(h, q3, deg3, W_self2, W_neigh2, b2, W_fc, b_fc)
  return (logits, h2)


# speedup vs baseline: 3.6496x; 1.0025x over previous
import functools

import jax
import jax.numpy as jnp
from jax import lax
from jax.experimental import pallas as pl
from jax.experimental.pallas import tpu as pltpu
from jax.experimental.pallas import tpu_sc as plsc

_NC = 2
_NS = 16
_CH = 80
_RD = 4


@functools.lru_cache(maxsize=None)
def _make_sc_agg(N, NF, EW, D, with_deg):
  ch, rd = (80, 4) if with_deg else (120, 3)
  ri = 2 * rd
  NCH = EW // ch
  tail = EW - NCH * ch
  rt = N // _NS
  assert N % (_NS * 8) == 0 and EW % 8 == 0 and tail % 8 == 0

  mesh = plsc.VectorSubcoreMesh(
      core_axis_name="c", subcore_axis_name="s",
      num_cores=_NC, num_subcores=_NS)

  out_type = [jax.ShapeDtypeStruct((_NC * N, D), jnp.float32)]
  scratch = [
      [pltpu.VMEM((ch,), jnp.int32) for _ in range(ri)],
      [pltpu.VMEM((ch,), jnp.int32) for _ in range(ri)],
      [pltpu.VMEM((ch, D), jnp.float32) for _ in range(rd)],
      [pltpu.SemaphoreType.DMA for _ in range(ri)],
      [pltpu.SemaphoreType.DMA for _ in range(rd)],
      [pltpu.SemaphoreType.DMA for _ in range(rd)],
      pltpu.VMEM_SHARED((N, D), jnp.float32),
      pltpu.VMEM((ch,), jnp.float32),
      [pltpu.VMEM((max(tail, 8),), jnp.int32) for _ in range(2)],
  ]
  if with_deg:
    out_type.append(jax.ShapeDtypeStruct((_NC * N,), jnp.float32))
    scratch.append(pltpu.VMEM_SHARED((N,), jnp.float32))
    scratch.append(pltpu.VMEM((ch,), jnp.float32))

  @functools.partial(
      pl.kernel,
      mesh=mesh,
      compiler_params=pltpu.CompilerParams(use_tc_tiling_on_sc=False),
      out_type=out_type,
      scratch_types=scratch,
  )
  def sc_agg(feat_hbm, src_hbm, dst_hbm, *refs):
    if with_deg:
      (out_hbm, deg_hbm, sbuf, dbuf, rows, isems, gsems, ssems, acc, zero_v,
       tbuf, dacc, ones_v) = refs
    else:
      out_hbm, sbuf, dbuf, rows, isems, gsems, ssems, acc, zero_v, tbuf = refs

    c = lax.axis_index("c")
    s = lax.axis_index("s")
    r0 = s * rt
    def zloop(j, carry):
      rows[0][j // (D // 16), pl.ds((j % (D // 16)) * 16, 16)] = (
          jnp.zeros((16,), jnp.float32))
      return carry
    lax.fori_loop(0, ch * D // 16, zloop, 0)
    for i in range(ch // 16):
      zero_v[pl.ds(i * 16, 16)] = jnp.zeros((16,), jnp.float32)
      if with_deg:
        ones_v[pl.ds(i * 16, 16)] = jnp.ones((16,), jnp.float32)
    nz = rt // ch
    for j in range(nz):
      pltpu.sync_copy(rows[0].at[pl.ds(0, ch)],
                      acc.at[pl.ds(r0 + j * ch, ch)])
      if with_deg:
        pltpu.sync_copy(zero_v, dacc.at[pl.ds(r0 + j * ch, ch)])
    rem = rt - nz * ch
    if rem:
      pltpu.sync_copy(rows[0].at[pl.ds(0, rem)],
                      acc.at[pl.ds(r0 + nz * ch, rem)])
      if with_deg:
        pltpu.sync_copy(zero_v.at[pl.ds(0, rem)],
                        dacc.at[pl.ds(r0 + nz * ch, rem)])
    plsc.subcore_barrier()

    wid = c * _NS + s

    def idx_copies(k, bi):
      return (pltpu.make_async_copy(src_hbm.at[wid, pl.ds(k * ch, ch)],
                                    sbuf[bi], isems[bi]),
              pltpu.make_async_copy(dst_hbm.at[wid, pl.ds(k * ch, ch)],
                                    dbuf[bi], isems[bi]))

    def gather_copy(br, bi):
      return pltpu.make_async_copy(feat_hbm.at[sbuf[bi]], rows[br], gsems[br])

    def scatter_start(br, bi):
      if with_deg:
        pltpu.async_copy(ones_v, dacc.at[dbuf[bi]], ssems[br], add=True)
      pltpu.async_copy(rows[br], acc.at[dbuf[bi]], ssems[br], add=True)

    def scatter_wait(br, bi):
      if with_deg:
        pltpu.make_async_copy(ones_v, dacc.at[dbuf[bi]], ssems[br]).wait()
      pltpu.make_async_copy(rows[br], acc.at[dbuf[bi]], ssems[br]).wait()

    for j in range(ri - 2):
      for cp in idx_copies(j, j):
        cp.start()
    for j in range(2):
      for cp in idx_copies(j, j):
        cp.wait()
      gather_copy(j, j).start()

    def turn(k, j):
      k = jnp.int32(k)
      @pl.when(k >= 2)
      def _():
        scatter_wait((j - 2) % rd, (j - 2) % ri)

      @pl.when(k + ri - 2 < NCH)
      def _():
        for cp in idx_copies(k + ri - 2, (j + ri - 2) % ri):
          cp.start()

      @pl.when(k + 2 < NCH)
      def _():
        for cp in idx_copies(k + 2, (j + 2) % ri):
          cp.wait()
        gather_copy((j + 2) % rd, (j + 2) % ri).start()

      gather_copy(j % rd, j % ri).wait()
      scatter_start(j % rd, j % ri)

    def body(o, carry):
      for j in range(ri):
        turn(ri * o + j, j)
      return carry

    lax.fori_loop(0, NCH // ri, body, 0)
    for k in range(NCH - NCH % ri, NCH):
      turn(k, k % ri)
    for k in (NCH - 2, NCH - 1):
      scatter_wait(k % rd, k % ri)
    if tail:
      pltpu.sync_copy(src_hbm.at[wid, pl.ds(NCH * ch, tail)], tbuf[0])
      pltpu.sync_copy(dst_hbm.at[wid, pl.ds(NCH * ch, tail)], tbuf[1])
      pltpu.async_copy(feat_hbm.at[tbuf[0]], rows[0].at[pl.ds(0, tail)],
                       gsems[0]).wait()
      if with_deg:
        pltpu.sync_copy(ones_v.at[pl.ds(0, tail)], dacc.at[tbuf[1]], add=True)
      pltpu.sync_copy(rows[0].at[pl.ds(0, tail)], acc.at[tbuf[1]], add=True)
    plsc.subcore_barrier()
    pltpu.sync_copy(acc.at[pl.ds(r0, rt)],
                    out_hbm.at[pl.ds(c * N + r0, rt)])
    if with_deg:
      pltpu.sync_copy(dacc.at[pl.ds(r0, rt)],
                      deg_hbm.at[pl.ds(c * N + r0, rt)])

  return sc_agg


def _tc_layer1(x, p3, deg3, w_self, w_neigh, b):
  n, d = x.shape
  bn = 1000
  assert n % bn == 0

  def body(x_ref, p0_ref, p1_ref, d0_ref, d1_ref, ws_ref, wn_ref, b_ref,
           h_ref):
    dg = d0_ref[0] + d1_ref[0]
    dinv = 1.0 / jnp.maximum(dg, 1.0)
    agg = (p0_ref[0] + p1_ref[0]) * dinv
    h = (jnp.dot(x_ref[...], ws_ref[...], preferred_element_type=jnp.float32)
         + jnp.dot(agg, wn_ref[...], preferred_element_type=jnp.float32)
         + b_ref[...])
    h_ref[...] = jnp.maximum(h, 0.0)

  return pl.pallas_call(
      body,
      grid=(n // bn,),
      in_specs=[
          pl.BlockSpec((bn, d), lambda i: (i, 0)),
          pl.BlockSpec((1, bn, d), lambda i: (0, i, 0)),
          pl.BlockSpec((1, bn, d), lambda i: (1, i, 0)),
          pl.BlockSpec((1, bn, 1), lambda i: (0, i, 0)),
          pl.BlockSpec((1, bn, 1), lambda i: (1, i, 0)),
          pl.BlockSpec((d, d), lambda i: (0, 0)),
          pl.BlockSpec((d, d), lambda i: (0, 0)),
          pl.BlockSpec((1, d), lambda i: (0, 0)),
      ],
      out_specs=pl.BlockSpec((bn, d), lambda i: (i, 0)),
      out_shape=jax.ShapeDtypeStruct((n, d), jnp.float32),
  )(x, p3, p3, deg3, deg3, w_self, w_neigh, b.reshape(1, d))


def _tc_layer2(h, q3, deg3, w_self, w_neigh, b, w_fc, b_fc):
  n, d = h.shape
  co = w_fc.shape[1]
  bn = 1000
  assert n % bn == 0

  def body(h_ref, q0_ref, q1_ref, d0_ref, d1_ref, ws_ref, wn_ref, b_ref,
           wfc_ref, bfc_ref, logits_ref, h2_ref):
    dg = d0_ref[0] + d1_ref[0]
    dinv = 1.0 / jnp.maximum(dg, 1.0)
    agg = (q0_ref[0] + q1_ref[0]) * dinv
    h2 = (jnp.dot(h_ref[...], ws_ref[...], preferred_element_type=jnp.float32)
          + jnp.dot(agg, wn_ref[...], preferred_element_type=jnp.float32)
          + b_ref[...])
    h2_ref[...] = h2
    logits_ref[...] = (
        jnp.dot(h2, wfc_ref[...], preferred_element_type=jnp.float32)
        + bfc_ref[...])

  return pl.pallas_call(
      body,
      grid=(n // bn,),
      in_specs=[
          pl.BlockSpec((bn, d), lambda i: (i, 0)),
          pl.BlockSpec((1, bn, d), lambda i: (0, i, 0)),
          pl.BlockSpec((1, bn, d), lambda i: (1, i, 0)),
          pl.BlockSpec((1, bn, 1), lambda i: (0, i, 0)),
          pl.BlockSpec((1, bn, 1), lambda i: (1, i, 0)),
          pl.BlockSpec((d, d), lambda i: (0, 0)),
          pl.BlockSpec((d, d), lambda i: (0, 0)),
          pl.BlockSpec((1, d), lambda i: (0, 0)),
          pl.BlockSpec((d, co), lambda i: (0, 0)),
          pl.BlockSpec((1, co), lambda i: (0, 0)),
      ],
      out_specs=[pl.BlockSpec((bn, co), lambda i: (i, 0)),
                 pl.BlockSpec((bn, d), lambda i: (i, 0))],
      out_shape=[jax.ShapeDtypeStruct((n, co), jnp.float32),
                 jax.ShapeDtypeStruct((n, d), jnp.float32)],
  )(h, q3, q3, deg3, deg3, w_self, w_neigh, b.reshape(1, d), w_fc,
    b_fc.reshape(1, co))


def kernel(x, edge_index, W_self1, W_neigh1, b1, W_self2, W_neigh2, b2,
           W_fc, b_fc):
  n, d = x.shape
  e = edge_index.shape[1]
  nwk = _NC * _NS
  n2 = ((n + _NS * 8 - 1) // (_NS * 8)) * (_NS * 8)

  assert e % (nwk * 8) == 0
  ew = e // nwk
  src = edge_index[0].astype(jnp.int32).reshape(nwk, ew)
  dst = edge_index[1].astype(jnp.int32).reshape(nwk, ew)

  p, deg = _make_sc_agg(n2, n, ew, d, True)(x, src, dst)
  p3 = p.reshape(_NC, n2, d)
  deg3 = deg.reshape(_NC, n2, 1)
  h = _tc_layer1(x, p3, deg3, W_self1, W_neigh1, b1)
  (q,) = _make_sc_agg(n2, n, ew, d, False)(h, src, dst)
  q3 = q.reshape(_NC, n2, d)
  logits, h2 = _tc_layer2(h, q3, deg3, W_self2, W_neigh2, b2, W_fc, b_fc)
  return (logits, h2)
